# initial kernel scaffold (unmeasured)
import jax
import jax.numpy as jnp
from jax import lax
from jax.experimental import pallas as pl
from jax.experimental.pallas import tpu as pltpu


def kernel(
    x,
):
    def body(*refs):
        pass

    out_shape = jax.ShapeDtypeStruct(..., jnp.float32)
    return pl.pallas_call(body, out_shape=out_shape)(...)


# baseline (device time: 19345 ns/iter reference)
import jax
import jax.numpy as jnp
from jax import lax
from jax.experimental import pallas as pl
from jax.experimental.pallas import tpu as pltpu

N_DEV = 4


def kernel(x):
    m_per, n_per = x.shape

    def body(x_ref, out_ref, comm_ref, send_sems, recv_sems):
        my = lax.axis_index("i")
        left = (my - 1) % N_DEV
        right = (my + 1) % N_DEV

        barrier_sem = pltpu.get_barrier_semaphore()
        for nbr in (left, right):
            pl.semaphore_signal(
                barrier_sem, inc=1,
                device_id=(nbr,), device_id_type=pl.DeviceIdType.MESH,
            )
        pl.semaphore_wait(barrier_sem, 2)

        xv = x_ref[:, :]
        m = jnp.max(xv, axis=1, keepdims=True)
        s = jnp.sum(jnp.exp(xv - m), axis=1, keepdims=True)

        comm_ref[0, :, 0:1] = m
        comm_ref[0, :, 1:2] = s

        M = m
        S = s
        for h in range(N_DEV - 1):
            rdma = pltpu.make_async_remote_copy(
                src_ref=comm_ref.at[h],
                dst_ref=comm_ref.at[h + 1],
                send_sem=send_sems.at[h],
                recv_sem=recv_sems.at[h],
                device_id=(right,),
                device_id_type=pl.DeviceIdType.MESH,
            )
            rdma.start()
            rdma.wait()
            m_j = comm_ref[h + 1, :, 0:1]
            s_j = comm_ref[h + 1, :, 1:2]
            newM = jnp.maximum(M, m_j)
            S = S * jnp.exp(M - newM) + s_j * jnp.exp(m_j - newM)
            M = newM

        out_ref[:, :] = jnp.exp(xv - M) / S

    return pl.pallas_call(
        body,
        out_shape=jax.ShapeDtypeStruct((m_per, n_per), x.dtype),
        in_specs=[pl.BlockSpec(memory_space=pltpu.VMEM)],
        out_specs=pl.BlockSpec(memory_space=pltpu.VMEM),
        scratch_shapes=[
            pltpu.VMEM((N_DEV, m_per, 2), jnp.float32),
            pltpu.SemaphoreType.DMA((N_DEV - 1,)),
            pltpu.SemaphoreType.DMA((N_DEV - 1,)),
        ],
        compiler_params=pltpu.CompilerParams(collective_id=0),
    )(x)


# device time: 12716 ns/iter; 1.5213x vs baseline; 1.5213x over previous
import jax
import jax.numpy as jnp
from jax import lax
from jax.experimental import pallas as pl
from jax.experimental.pallas import tpu as pltpu

N_DEV = 4


def kernel(x):
    m_per, n_per = x.shape

    def body(x_ref, out_ref, comm_ref, send_sems, recv_sems):
        my = lax.axis_index("i")

        barrier_sem = pltpu.get_barrier_semaphore()
        for d in range(1, N_DEV):
            pl.semaphore_signal(
                barrier_sem, inc=1,
                device_id=((my + d) % N_DEV,),
                device_id_type=pl.DeviceIdType.MESH,
            )

        xv = x_ref[:, :]
        m = jnp.max(xv, axis=1, keepdims=True)
        e = jnp.exp(xv - m)
        s = jnp.sum(e, axis=1, keepdims=True)
        out_ref[:, :] = e
        comm_ref[0, :, 0:1] = m
        comm_ref[0, :, 1:2] = s

        pl.semaphore_wait(barrier_sem, N_DEV - 1)

        rdmas = []
        for d in range(1, N_DEV):
            rdma = pltpu.make_async_remote_copy(
                src_ref=comm_ref.at[0],
                dst_ref=comm_ref.at[d],
                send_sem=send_sems.at[d - 1],
                recv_sem=recv_sems.at[d - 1],
                device_id=((my + d) % N_DEV,),
                device_id_type=pl.DeviceIdType.MESH,
            )
            rdma.start()
            rdmas.append(rdma)

        M = m
        S = s
        for d in range(1, N_DEV):
            rdmas[d - 1].wait_recv()
            m_j = comm_ref[d, :, 0:1]
            s_j = comm_ref[d, :, 1:2]
            newM = jnp.maximum(M, m_j)
            S = S * jnp.exp(M - newM) + s_j * jnp.exp(m_j - newM)
            M = newM
        for r in rdmas:
            r.wait_send()

        out_ref[:, :] = out_ref[:, :] * (jnp.exp(m - M) / S)

    return pl.pallas_call(
        body,
        out_shape=jax.ShapeDtypeStruct((m_per, n_per), x.dtype),
        in_specs=[pl.BlockSpec(memory_space=pltpu.VMEM)],
        out_specs=pl.BlockSpec(memory_space=pltpu.VMEM),
        scratch_shapes=[
            pltpu.VMEM((N_DEV, m_per, 2), jnp.float32),
            pltpu.SemaphoreType.DMA((N_DEV - 1,)),
            pltpu.SemaphoreType.DMA((N_DEV - 1,)),
        ],
        compiler_params=pltpu.CompilerParams(collective_id=0),
    )(x)


# device time: 8000 ns/iter; 2.4181x vs baseline; 1.5895x over previous
import jax
import jax.numpy as jnp
from jax import lax
from jax.experimental import pallas as pl
from jax.experimental.pallas import tpu as pltpu

N_DEV = 4


def kernel(x):
    m_per, n_per = x.shape
    rows_t = m_per // 128

    def body(x_ref, out_ref, comm_ref, send_sems, recv_sems):
        my = lax.axis_index("i")

        barrier_sem = pltpu.get_barrier_semaphore()
        for d in range(1, N_DEV):
            pl.semaphore_signal(
                barrier_sem, inc=1,
                device_id=((my + d) % N_DEV,),
                device_id_type=pl.DeviceIdType.MESH,
            )

        xv = x_ref[:, :]
        m = jnp.max(xv, axis=1, keepdims=True)
        e = jnp.exp(xv - m)
        s = jnp.sum(e, axis=1, keepdims=True)
        comm_ref[0, 0:rows_t, :] = jnp.reshape(m, (rows_t, 128))
        comm_ref[0, rows_t:2 * rows_t, :] = jnp.reshape(s, (rows_t, 128))

        pl.semaphore_wait(barrier_sem, N_DEV - 1)

        rdmas = []
        for d in range(1, N_DEV):
            rdma = pltpu.make_async_remote_copy(
                src_ref=comm_ref.at[0],
                dst_ref=comm_ref.at[d],
                send_sem=send_sems.at[d - 1],
                recv_sem=recv_sems.at[d - 1],
                device_id=((my + d) % N_DEV,),
                device_id_type=pl.DeviceIdType.MESH,
            )
            rdma.start()
            rdmas.append(rdma)

        out_ref[:, :] = e

        M = jnp.reshape(m, (rows_t, 128))
        S = jnp.reshape(s, (rows_t, 128))
        for d in range(1, N_DEV):
            rdmas[d - 1].wait_recv()
            m_j = comm_ref[d, 0:rows_t, :]
            s_j = comm_ref[d, rows_t:2 * rows_t, :]
            newM = jnp.maximum(M, m_j)
            S = S * jnp.exp(M - newM) + s_j * jnp.exp(m_j - newM)
            M = newM
        for r in rdmas:
            r.wait_send()

        scale = jnp.exp(m - jnp.reshape(M, (m_per, 1))) / jnp.reshape(
            S, (m_per, 1)
        )
        out_ref[:, :] = out_ref[:, :] * scale

    return pl.pallas_call(
        body,
        out_shape=jax.ShapeDtypeStruct((m_per, n_per), x.dtype),
        in_specs=[pl.BlockSpec(memory_space=pltpu.VMEM)],
        out_specs=pl.BlockSpec(memory_space=pltpu.VMEM),
        scratch_shapes=[
            pltpu.VMEM((N_DEV, 2 * m_per // 128, 128), jnp.float32),
            pltpu.SemaphoreType.DMA((N_DEV - 1,)),
            pltpu.SemaphoreType.DMA((N_DEV - 1,)),
        ],
        compiler_params=pltpu.CompilerParams(collective_id=0),
    )(x)


# device time: 7216 ns/iter; 2.6808x vs baseline; 1.1086x over previous
import jax
import jax.numpy as jnp
from jax import lax
from jax.experimental import pallas as pl
from jax.experimental.pallas import tpu as pltpu

N_DEV = 4
B = 2


def _unpack_col(packed, rows):
    rt = rows // 128
    row_i = lax.broadcasted_iota(jnp.int32, (rows, 128), 0)
    lane_i = lax.broadcasted_iota(jnp.int32, (rows, 128), 1)
    tmp = jnp.broadcast_to(packed[0:1, :], (rows, 128))
    for t in range(1, rt):
        tmp = jnp.where(
            row_i >= t * 128,
            jnp.broadcast_to(packed[t:t + 1, :], (rows, 128)),
            tmp,
        )
    sel = jnp.where(lane_i == row_i % 128, tmp, 0.0)
    return jnp.sum(sel, axis=1, keepdims=True)


def kernel(x):
    m_per, n_per = x.shape
    rb = m_per // B
    rt = rb // 128

    def body(x_ref, out_ref, comm_ref, send_sems, recv_sems):
        my = lax.axis_index("i")
        wait_order = (1, 3, 2)

        barrier_sem = pltpu.get_barrier_semaphore()
        for d in range(1, N_DEV):
            pl.semaphore_signal(
                barrier_sem, inc=1,
                device_id=((my + d) % N_DEV,),
                device_id_type=pl.DeviceIdType.MESH,
            )

        rdmas = [None] * (B * N_DEV)

        for b in range(B):
            xb = x_ref[b * rb:(b + 1) * rb, :]
            m = jnp.max(xb, axis=1, keepdims=True)
            e = jnp.exp(xb - m)
            s = jnp.sum(e, axis=1, keepdims=True)
            comm_ref[b, 0, 0:rt, :] = jnp.reshape(m, (rt, 128))
            comm_ref[b, 0, rt:2 * rt, :] = jnp.reshape(s, (rt, 128))
            if b == 0:
                pl.semaphore_wait(barrier_sem, N_DEV - 1)
            for d in range(1, N_DEV):
                rdma = pltpu.make_async_remote_copy(
                    src_ref=comm_ref.at[b, 0],
                    dst_ref=comm_ref.at[b, d],
                    send_sem=send_sems.at[b, d - 1],
                    recv_sem=recv_sems.at[b, d - 1],
                    device_id=((my + d) % N_DEV,),
                    device_id_type=pl.DeviceIdType.MESH,
                )
                rdma.start()
                rdmas[b * N_DEV + d] = rdma
            out_ref[b * rb:(b + 1) * rb, :] = e

        for b in range(B):
            m_pk = comm_ref[b, 0, 0:rt, :]
            M = m_pk
            S = comm_ref[b, 0, rt:2 * rt, :]
            for d in wait_order:
                rdmas[b * N_DEV + d].wait_recv()
                m_j = comm_ref[b, d, 0:rt, :]
                s_j = comm_ref[b, d, rt:2 * rt, :]
                newM = jnp.maximum(M, m_j)
                S = S * jnp.exp(M - newM) + s_j * jnp.exp(m_j - newM)
                M = newM
            scale = _unpack_col(jnp.exp(m_pk - M) / S, rb)
            out_ref[b * rb:(b + 1) * rb, :] = (
                out_ref[b * rb:(b + 1) * rb, :] * scale
            )

        for b in range(B):
            for d in range(1, N_DEV):
                rdmas[b * N_DEV + d].wait_send()

    return pl.pallas_call(
        body,
        out_shape=jax.ShapeDtypeStruct((m_per, n_per), x.dtype),
        in_specs=[pl.BlockSpec(memory_space=pltpu.VMEM)],
        out_specs=pl.BlockSpec(memory_space=pltpu.VMEM),
        scratch_shapes=[
            pltpu.VMEM((B, N_DEV, 2 * rt, 128), jnp.float32),
            pltpu.SemaphoreType.DMA((B, N_DEV - 1)),
            pltpu.SemaphoreType.DMA((B, N_DEV - 1)),
        ],
        compiler_params=pltpu.CompilerParams(collective_id=0),
    )(x)
